# baseline (speedup 1.0000x reference)
import functools

import jax
import jax.numpy as jnp
from jax import lax
from jax.experimental import pallas as pl
from jax.experimental.pallas import tpu as pltpu
from jax.experimental.pallas import tpu_sc as plsc

B = 16384
EMB = 16
M = 128

_NC = 2
_NS = 16
_NW = _NC * _NS
_BPW = B // _NW

_sc_mesh = plsc.VectorSubcoreMesh(core_axis_name="c", subcore_axis_name="s")


@functools.partial(
    pl.kernel,
    out_type=(
        jax.ShapeDtypeStruct((B, EMB), jnp.float32),
        jax.ShapeDtypeStruct((B, EMB), jnp.float32),
    ),
    mesh=_sc_mesh,
    scratch_types=[
        pltpu.VMEM((_BPW,), jnp.int32),
        pltpu.VMEM((_BPW,), jnp.int32),
        pltpu.VMEM((_BPW, EMB), jnp.float32),
        pltpu.VMEM((_BPW, EMB), jnp.float32),
        pltpu.SemaphoreType.DMA,
        pltpu.SemaphoreType.DMA,
    ],
    compiler_params=pltpu.CompilerParams(use_tc_tiling_on_sc=False),
)
def _sc_gather(user_hbm, movie_hbm, uid_hbm, mid_hbm, ueb_hbm, meb_hbm,
               uidx_v, midx_v, urows_v, mrows_v, usem, msem):
    wid = lax.axis_index("s") * _NC + lax.axis_index("c")
    base = wid * _BPW
    pltpu.sync_copy(uid_hbm.at[pl.ds(base, _BPW)], uidx_v)
    pltpu.sync_copy(mid_hbm.at[pl.ds(base, _BPW)], midx_v)
    ucp = pltpu.async_copy(user_hbm.at[uidx_v], urows_v, usem)
    mcp = pltpu.async_copy(movie_hbm.at[midx_v], mrows_v, msem)
    ucp.wait()
    mcp.wait()
    pltpu.sync_copy(urows_v, ueb_hbm.at[pl.ds(base, _BPW)])
    pltpu.sync_copy(mrows_v, meb_hbm.at[pl.ds(base, _BPW)])


_BLK = 2048


def _mlp_body(u_ref, m_ref, w1u_ref, w1m_ref, b1_ref, w2_ref, b2_ref,
              w3_ref, b3_ref, o_ref):
    h1 = (jnp.dot(u_ref[...], w1u_ref[...], preferred_element_type=jnp.float32)
          + jnp.dot(m_ref[...], w1m_ref[...], preferred_element_type=jnp.float32)
          + b1_ref[...])
    h1 = jnp.maximum(h1, 0.0)
    h2 = jnp.maximum(
        jnp.dot(h1, w2_ref[...], preferred_element_type=jnp.float32)
        + b2_ref[...], 0.0)
    o_ref[...] = (jnp.dot(h2, w3_ref[...], preferred_element_type=jnp.float32)
                  + b3_ref[...])


_mlp = pl.pallas_call(
    _mlp_body,
    grid=(B // _BLK,),
    in_specs=[
        pl.BlockSpec((_BLK, EMB), lambda i: (i, 0)),
        pl.BlockSpec((_BLK, EMB), lambda i: (i, 0)),
        pl.BlockSpec((EMB, M), lambda i: (0, 0)),
        pl.BlockSpec((EMB, M), lambda i: (0, 0)),
        pl.BlockSpec((1, M), lambda i: (0, 0)),
        pl.BlockSpec((M, M // 2), lambda i: (0, 0)),
        pl.BlockSpec((1, M // 2), lambda i: (0, 0)),
        pl.BlockSpec((M // 2, 1), lambda i: (0, 0)),
        pl.BlockSpec((1, 1), lambda i: (0, 0)),
    ],
    out_specs=pl.BlockSpec((_BLK, 1), lambda i: (i, 0)),
    out_shape=jax.ShapeDtypeStruct((B, 1), jnp.float32),
)


def kernel(userId, movieId, user_table, movie_table, W1, b1, W2, b2, W3, b3):
    ueb, meb = _sc_gather(user_table, movie_table, userId, movieId)
    w1t = W1.T
    return _mlp(ueb, meb, w1t[:EMB], w1t[EMB:], b1.reshape(1, M),
                W2.T, b2.reshape(1, M // 2), W3.T, b3.reshape(1, 1))
